# baseline (speedup 1.0000x reference)
import functools

import jax
import jax.numpy as jnp
from jax import lax
from jax.experimental import pallas as pl
from jax.experimental.pallas import tpu as pltpu
from jax.experimental.pallas import tpu_sc as plsc

_K = 12
_CT = 512
_CHC = 32


def _weights_body(thres_ref, rout_ref, dout_ref, w_ref):
    ft = thres_ref[0]
    bt = thres_ref[1]

    def probs(o):
        x0, x1 = o[:, 0, :], o[:, 1, :]
        m = jnp.maximum(x0, x1)
        e0 = jnp.exp(x0 - m)
        e1 = jnp.exp(x1 - m)
        s = e0 + e1
        return e1 / s, e0 / s

    rfg, rbg = probs(rout_ref[...])
    dfg, dbg = probs(dout_ref[...])

    mfg = (rfg > ft) & (dfg > ft)
    mbg = (rbg > bt) & (dbg > bt)
    cfg = jnp.sum(mfg.astype(jnp.float32), axis=1, keepdims=True)
    cbg = jnp.sum(mbg.astype(jnp.float32), axis=1, keepdims=True)
    wmfg = mfg.astype(jnp.float32) / jnp.maximum(cfg, 1.0)
    wmbg = mbg.astype(jnp.float32) / jnp.maximum(cbg, 1.0)

    scores = jnp.concatenate([rfg, dfg, rbg, dbg], axis=0)
    iota = lax.broadcasted_iota(jnp.int32, scores.shape, 1)

    def topk_step(_, carry):
        vals, ind = carry
        m = jnp.max(vals, axis=1, keepdims=True)
        cand = jnp.where(vals == m, iota, jnp.int32(2**30))
        imin = jnp.min(cand, axis=1, keepdims=True)
        sel = iota == imin
        return jnp.where(sel, -1.0, vals), ind + sel.astype(jnp.float32)

    _, ind = lax.fori_loop(0, _K, topk_step,
                           (scores, jnp.zeros_like(scores)))
    ind = ind * (1.0 / _K)

    bs = rfg.shape[0]
    use_fg = cfg > 0.0
    use_bg = cbg > 0.0
    w_ref[:, 0, :] = jnp.where(use_fg, wmfg, ind[0 * bs:1 * bs])
    w_ref[:, 1, :] = jnp.where(use_fg, wmfg, ind[1 * bs:2 * bs])
    w_ref[:, 2, :] = jnp.where(use_bg, wmbg, ind[2 * bs:3 * bs])
    w_ref[:, 3, :] = jnp.where(use_bg, wmbg, ind[3 * bs:4 * bs])


def _tc_dense_body(rfea_ref, dfea_ref, w_ref,
                   rfg_ref, rbg_ref, dfg_ref, dbg_ref):
    fr = rfea_ref[0]
    fd = dfea_ref[0]
    w = w_ref[0]
    rfg_ref[0, 0] = jnp.sum(fr * w[0:1], axis=1)
    dfg_ref[0, 0] = jnp.sum(fd * w[1:2], axis=1)
    rbg_ref[0, 0] = jnp.sum(fr * w[2:3], axis=1)
    dbg_ref[0, 0] = jnp.sum(fd * w[3:4], axis=1)


def _sc_dense_body(rfea, dfea, w_hbm, out_hbm, wbuf, bufs, obuf, wsem, sem):
    wid = lax.axis_index("s") * 2 + lax.axis_index("c")
    b = wid // 4
    g = wid % 4
    per_tile = out_hbm.shape[2] // 4
    cbase = _CT + g * per_tile
    nch = per_tile // _CHC

    pltpu.async_copy(w_hbm.at[b], wbuf, wsem).wait()

    for f, fea in enumerate((rfea, dfea)):
        w_fg = 0 if f == 0 else 1
        w_bg = 2 if f == 0 else 3
        copies = [
            pltpu.async_copy(
                fea.at[b, pl.ds(cbase + k * _CHC, _CHC)],
                bufs.at[k], sem.at[k])
            for k in range(2)
        ]
        lane = lax.iota(jnp.int32, 16)
        for k in range(nch):
            slot = k % 2
            copies[slot].wait()
            for half in range(_CHC // 16):
                res_fg = jnp.zeros((16,), jnp.float32)
                res_bg = jnp.zeros((16,), jnp.float32)
                for rg in range(2):
                    rbase = half * 16 + rg * 8

                    def grp(l4, accs, _slot=slot, _rb=rbase,
                            _wf=w_fg, _wb=w_bg):
                        for u in range(4):
                            l = l4 * 4 + u
                            wf = wbuf[_wf, pl.ds(l * 16, 16)]
                            wb = wbuf[_wb, pl.ds(l * 16, 16)]
                            fgs, bgs = [], []
                            for rr in range(8):
                                v = bufs[_slot, _rb + rr,
                                         pl.ds(l * 16, 16)]
                                fgs.append(accs[rr] + v * wf)
                                bgs.append(accs[8 + rr] + v * wb)
                            accs = tuple(fgs) + tuple(bgs)
                        return accs

                    zero = tuple(jnp.zeros((16,), jnp.float32)
                                 for _ in range(16))
                    accs = lax.fori_loop(0, 16, grp, zero)

                    def lanesum(v):
                        for s in (8, 4, 2, 1):
                            perm = lane ^ s
                            v = v + v.at[perm].get(
                                mode="promise_in_bounds")
                        return v

                    for rr in range(8):
                        sel = lane == (rg * 8 + rr)
                        res_fg = jnp.where(sel, lanesum(accs[rr]), res_fg)
                        res_bg = jnp.where(sel, lanesum(accs[8 + rr]),
                                           res_bg)
                cb = k * _CHC + half * 16
                obuf[pl.ds(w_fg * per_tile + cb, 16)] = res_fg
                obuf[pl.ds(w_bg * per_tile + cb, 16)] = res_bg
            nk = k + 2
            if nk < nch:
                copies[slot] = pltpu.async_copy(
                    fea.at[b, pl.ds(cbase + nk * _CHC, _CHC)],
                    bufs.at[slot], sem.at[slot])

    pltpu.sync_copy(obuf, out_hbm.at[b, g])


@jax.jit
def _run(res_fea, dinov2_fea, res_out, dinov2_out, thres):
    bs, C = res_fea.shape[0], res_fea.shape[1]
    HW = res_fea.shape[2] * res_fea.shape[3]
    scc = C - _CT
    rfea = res_fea.reshape(bs, C, HW)
    dfea = dinov2_fea.reshape(bs, C, HW)
    rout = res_out.reshape(bs, 2, HW)
    dout = dinov2_out.reshape(bs, 2, HW)

    w = pl.pallas_call(
        _weights_body,
        in_specs=[
            pl.BlockSpec(memory_space=pltpu.SMEM),
            pl.BlockSpec((bs, 2, HW), lambda: (0, 0, 0)),
            pl.BlockSpec((bs, 2, HW), lambda: (0, 0, 0)),
        ],
        out_specs=pl.BlockSpec((bs, 4, HW), lambda: (0, 0, 0)),
        out_shape=jax.ShapeDtypeStruct((bs, 4, HW), jnp.float32),
    )(thres, rout, dout)

    per_tile = scc // 4
    sc_out = pl.kernel(
        _sc_dense_body,
        out_type=jax.ShapeDtypeStruct((bs, 4, 4 * per_tile), jnp.float32),
        mesh=plsc.VectorSubcoreMesh(core_axis_name="c",
                                    subcore_axis_name="s"),
        scratch_types=[
            pltpu.VMEM((4, HW), jnp.float32),
            pltpu.VMEM((2, _CHC, HW), jnp.float32),
            pltpu.VMEM((4 * per_tile,), jnp.float32),
            pltpu.SemaphoreType.DMA,
            pltpu.SemaphoreType.DMA((2,)),
        ],
        compiler_params=pltpu.CompilerParams(skip_device_barrier=True),
    )(rfea, dfea, w)
    sc_cls = sc_out.reshape(bs, 4, 4, per_tile).transpose(0, 2, 1, 3)
    sc_cls = sc_cls.reshape(bs, 4, scc)

    tc_outs = pl.pallas_call(
        _tc_dense_body,
        grid=(bs,),
        in_specs=[
            pl.BlockSpec((1, _CT, HW), lambda b: (b, 0, 0)),
            pl.BlockSpec((1, _CT, HW), lambda b: (b, 0, 0)),
            pl.BlockSpec((1, 4, HW), lambda b: (b, 0, 0)),
        ],
        out_specs=[pl.BlockSpec((1, 1, _CT), lambda b: (b, 0, 0))] * 4,
        out_shape=[jax.ShapeDtypeStruct((bs, 1, _CT), jnp.float32)] * 4,
    )(rfea, dfea, w)

    shape = (bs, C, 1, 1)
    order = (0, 2, 1, 3)
    return tuple(
        jnp.concatenate([tc_outs[i][:, 0, :], sc_cls[:, order[i], :]],
                        axis=1).reshape(shape)
        for i in range(4))


def kernel(res_fea, dinov2_fea, res_out, dinov2_out, fg_thres, bg_thres):
    thres = jnp.stack([jnp.asarray(fg_thres, jnp.float32),
                       jnp.asarray(bg_thres, jnp.float32)])
    return _run(res_fea, dinov2_fea, res_out, dinov2_out, thres)


# speedup vs baseline: 1.0008x; 1.0008x over previous
import functools

import jax
import jax.numpy as jnp
from jax import lax
from jax.experimental import pallas as pl
from jax.experimental.pallas import tpu as pltpu
from jax.experimental.pallas import tpu_sc as plsc

_K = 12
_CT = 640
_CHC = 32


def _weights_body(thres_ref, rout_ref, dout_ref, w_ref):
    ft = thres_ref[0]
    bt = thres_ref[1]

    def probs(o):
        x0, x1 = o[:, 0, :], o[:, 1, :]
        m = jnp.maximum(x0, x1)
        e0 = jnp.exp(x0 - m)
        e1 = jnp.exp(x1 - m)
        s = e0 + e1
        return e1 / s, e0 / s

    rfg, rbg = probs(rout_ref[...])
    dfg, dbg = probs(dout_ref[...])

    mfg = (rfg > ft) & (dfg > ft)
    mbg = (rbg > bt) & (dbg > bt)
    cfg = jnp.sum(mfg.astype(jnp.float32), axis=1, keepdims=True)
    cbg = jnp.sum(mbg.astype(jnp.float32), axis=1, keepdims=True)
    wmfg = mfg.astype(jnp.float32) / jnp.maximum(cfg, 1.0)
    wmbg = mbg.astype(jnp.float32) / jnp.maximum(cbg, 1.0)

    scores = jnp.concatenate([rfg, dfg, rbg, dbg], axis=0)
    iota = lax.broadcasted_iota(jnp.int32, scores.shape, 1)

    def topk_step(_, carry):
        vals, ind = carry
        m = jnp.max(vals, axis=1, keepdims=True)
        cand = jnp.where(vals == m, iota, jnp.int32(2**30))
        imin = jnp.min(cand, axis=1, keepdims=True)
        sel = iota == imin
        return jnp.where(sel, -1.0, vals), ind + sel.astype(jnp.float32)

    _, ind = lax.fori_loop(0, _K, topk_step,
                           (scores, jnp.zeros_like(scores)))
    ind = ind * (1.0 / _K)

    bs = rfg.shape[0]
    use_fg = cfg > 0.0
    use_bg = cbg > 0.0
    w_ref[:, 0, :] = jnp.where(use_fg, wmfg, ind[0 * bs:1 * bs])
    w_ref[:, 1, :] = jnp.where(use_fg, wmfg, ind[1 * bs:2 * bs])
    w_ref[:, 2, :] = jnp.where(use_bg, wmbg, ind[2 * bs:3 * bs])
    w_ref[:, 3, :] = jnp.where(use_bg, wmbg, ind[3 * bs:4 * bs])


def _tc_dense_body(rfea_ref, dfea_ref, w_ref,
                   rfg_ref, rbg_ref, dfg_ref, dbg_ref):
    fr = rfea_ref[0]
    fd = dfea_ref[0]
    w = w_ref[0]
    rfg_ref[0, 0] = jnp.sum(fr * w[0:1], axis=1)
    dfg_ref[0, 0] = jnp.sum(fd * w[1:2], axis=1)
    rbg_ref[0, 0] = jnp.sum(fr * w[2:3], axis=1)
    dbg_ref[0, 0] = jnp.sum(fd * w[3:4], axis=1)


def _sc_dense_body(rfea, dfea, w_hbm, out_hbm, wbuf, bufs, obuf, wsem, sem):
    wid = lax.axis_index("s") * 2 + lax.axis_index("c")
    b = wid // 4
    g = wid % 4
    per_tile = out_hbm.shape[2] // 4
    cbase = _CT + g * per_tile
    nch = per_tile // _CHC

    pltpu.async_copy(w_hbm.at[b], wbuf, wsem).wait()

    for f, fea in enumerate((rfea, dfea)):
        w_fg = 0 if f == 0 else 1
        w_bg = 2 if f == 0 else 3
        copies = [
            pltpu.async_copy(
                fea.at[b, pl.ds(cbase + k * _CHC, _CHC)],
                bufs.at[k], sem.at[k])
            for k in range(min(2, nch))
        ]
        lane = lax.iota(jnp.int32, 16)
        for k in range(nch):
            slot = k % 2
            copies[slot].wait()
            for half in range(_CHC // 16):
                res_fg = jnp.zeros((16,), jnp.float32)
                res_bg = jnp.zeros((16,), jnp.float32)
                for rg in range(2):
                    rbase = half * 16 + rg * 8

                    def grp(l4, accs, _slot=slot, _rb=rbase,
                            _wf=w_fg, _wb=w_bg):
                        for u in range(4):
                            l = l4 * 4 + u
                            wf = wbuf[_wf, pl.ds(l * 16, 16)]
                            wb = wbuf[_wb, pl.ds(l * 16, 16)]
                            fgs, bgs = [], []
                            for rr in range(8):
                                v = bufs[_slot, _rb + rr,
                                         pl.ds(l * 16, 16)]
                                fgs.append(accs[rr] + v * wf)
                                bgs.append(accs[8 + rr] + v * wb)
                            accs = tuple(fgs) + tuple(bgs)
                        return accs

                    zero = tuple(jnp.zeros((16,), jnp.float32)
                                 for _ in range(16))
                    accs = lax.fori_loop(0, 16, grp, zero)

                    def lanesum(v):
                        for s in (8, 4, 2, 1):
                            perm = lane ^ s
                            v = v + v.at[perm].get(
                                mode="promise_in_bounds")
                        return v

                    for rr in range(8):
                        sel = lane == (rg * 8 + rr)
                        res_fg = jnp.where(sel, lanesum(accs[rr]), res_fg)
                        res_bg = jnp.where(sel, lanesum(accs[8 + rr]),
                                           res_bg)
                cb = k * _CHC + half * 16
                obuf[pl.ds(w_fg * per_tile + cb, 16)] = res_fg
                obuf[pl.ds(w_bg * per_tile + cb, 16)] = res_bg
            nk = k + 2
            if nk < nch:
                copies[slot] = pltpu.async_copy(
                    fea.at[b, pl.ds(cbase + nk * _CHC, _CHC)],
                    bufs.at[slot], sem.at[slot])

    pltpu.sync_copy(obuf, out_hbm.at[b, g])


@jax.jit
def _run(res_fea, dinov2_fea, res_out, dinov2_out, thres):
    bs, C = res_fea.shape[0], res_fea.shape[1]
    HW = res_fea.shape[2] * res_fea.shape[3]
    scc = C - _CT
    rfea = res_fea.reshape(bs, C, HW)
    dfea = dinov2_fea.reshape(bs, C, HW)
    rout = res_out.reshape(bs, 2, HW)
    dout = dinov2_out.reshape(bs, 2, HW)

    w = pl.pallas_call(
        _weights_body,
        in_specs=[
            pl.BlockSpec(memory_space=pltpu.SMEM),
            pl.BlockSpec((bs, 2, HW), lambda: (0, 0, 0)),
            pl.BlockSpec((bs, 2, HW), lambda: (0, 0, 0)),
        ],
        out_specs=pl.BlockSpec((bs, 4, HW), lambda: (0, 0, 0)),
        out_shape=jax.ShapeDtypeStruct((bs, 4, HW), jnp.float32),
    )(thres, rout, dout)

    per_tile = scc // 4
    sc_out = pl.kernel(
        _sc_dense_body,
        out_type=jax.ShapeDtypeStruct((bs, 4, 4 * per_tile), jnp.float32),
        mesh=plsc.VectorSubcoreMesh(core_axis_name="c",
                                    subcore_axis_name="s"),
        scratch_types=[
            pltpu.VMEM((4, HW), jnp.float32),
            pltpu.VMEM((2, _CHC, HW), jnp.float32),
            pltpu.VMEM((4 * per_tile,), jnp.float32),
            pltpu.SemaphoreType.DMA,
            pltpu.SemaphoreType.DMA((2,)),
        ],
        compiler_params=pltpu.CompilerParams(skip_device_barrier=True),
    )(rfea, dfea, w)
    sc_cls = sc_out.reshape(bs, 4, 4, per_tile).transpose(0, 2, 1, 3)
    sc_cls = sc_cls.reshape(bs, 4, scc)

    tc_outs = pl.pallas_call(
        _tc_dense_body,
        grid=(bs,),
        in_specs=[
            pl.BlockSpec((1, _CT, HW), lambda b: (b, 0, 0)),
            pl.BlockSpec((1, _CT, HW), lambda b: (b, 0, 0)),
            pl.BlockSpec((1, 4, HW), lambda b: (b, 0, 0)),
        ],
        out_specs=[pl.BlockSpec((1, 1, _CT), lambda b: (b, 0, 0))] * 4,
        out_shape=[jax.ShapeDtypeStruct((bs, 1, _CT), jnp.float32)] * 4,
    )(rfea, dfea, w)

    shape = (bs, C, 1, 1)
    order = (0, 2, 1, 3)
    return tuple(
        jnp.concatenate([tc_outs[i][:, 0, :], sc_cls[:, order[i], :]],
                        axis=1).reshape(shape)
        for i in range(4))


def kernel(res_fea, dinov2_fea, res_out, dinov2_out, fg_thres, bg_thres):
    thres = jnp.stack([jnp.asarray(fg_thres, jnp.float32),
                       jnp.asarray(bg_thres, jnp.float32)])
    return _run(res_fea, dinov2_fea, res_out, dinov2_out, thres)


# speedup vs baseline: 1.2532x; 1.2522x over previous
import jax
import jax.numpy as jnp
from jax import lax
from jax.experimental import pallas as pl
from jax.experimental.pallas import tpu as pltpu

_K = 12


def _weights_body(thres_ref, rout_ref, dout_ref, w_ref):
    ft = thres_ref[0]
    bt = thres_ref[1]

    def probs(o):
        x0, x1 = o[:, 0, :], o[:, 1, :]
        m = jnp.maximum(x0, x1)
        e0 = jnp.exp(x0 - m)
        e1 = jnp.exp(x1 - m)
        s = e0 + e1
        return e1 / s, e0 / s

    rfg, rbg = probs(rout_ref[...])
    dfg, dbg = probs(dout_ref[...])

    mfg = (rfg > ft) & (dfg > ft)
    mbg = (rbg > bt) & (dbg > bt)
    cfg = jnp.sum(mfg.astype(jnp.float32), axis=1, keepdims=True)
    cbg = jnp.sum(mbg.astype(jnp.float32), axis=1, keepdims=True)
    wmfg = mfg.astype(jnp.float32) / jnp.maximum(cfg, 1.0)
    wmbg = mbg.astype(jnp.float32) / jnp.maximum(cbg, 1.0)

    scores = jnp.concatenate([rfg, dfg, rbg, dbg], axis=0)
    iota = lax.broadcasted_iota(jnp.int32, scores.shape, 1)

    def topk_step(_, carry):
        vals, ind = carry
        m = jnp.max(vals, axis=1, keepdims=True)
        cand = jnp.where(vals == m, iota, jnp.int32(2**30))
        imin = jnp.min(cand, axis=1, keepdims=True)
        sel = iota == imin
        return jnp.where(sel, -1.0, vals), ind + sel.astype(jnp.float32)

    _, ind = lax.fori_loop(0, _K, topk_step,
                           (scores, jnp.zeros_like(scores)))
    ind = ind * (1.0 / _K)

    bs = rfg.shape[0]
    use_fg = cfg > 0.0
    use_bg = cbg > 0.0
    w_ref[:, 0, :] = jnp.where(use_fg, wmfg, ind[0 * bs:1 * bs])
    w_ref[:, 1, :] = jnp.where(use_fg, wmfg, ind[1 * bs:2 * bs])
    w_ref[:, 2, :] = jnp.where(use_bg, wmbg, ind[2 * bs:3 * bs])
    w_ref[:, 3, :] = jnp.where(use_bg, wmbg, ind[3 * bs:4 * bs])


def _dense_body(rfea_ref, dfea_ref, w_ref,
                rfg_ref, rbg_ref, dfg_ref, dbg_ref):
    fr = rfea_ref[0]
    fd = dfea_ref[0]
    w = w_ref[0]
    rfg_ref[0, 0] = jnp.sum(fr * w[0:1], axis=1)
    dfg_ref[0, 0] = jnp.sum(fd * w[1:2], axis=1)
    rbg_ref[0, 0] = jnp.sum(fr * w[2:3], axis=1)
    dbg_ref[0, 0] = jnp.sum(fd * w[3:4], axis=1)


@jax.jit
def _run(res_fea, dinov2_fea, res_out, dinov2_out, thres):
    bs, C = res_fea.shape[0], res_fea.shape[1]
    HW = res_fea.shape[2] * res_fea.shape[3]
    rfea = res_fea.reshape(bs, C, HW)
    dfea = dinov2_fea.reshape(bs, C, HW)
    rout = res_out.reshape(bs, 2, HW)
    dout = dinov2_out.reshape(bs, 2, HW)

    w = pl.pallas_call(
        _weights_body,
        in_specs=[
            pl.BlockSpec(memory_space=pltpu.SMEM),
            pl.BlockSpec((bs, 2, HW), lambda: (0, 0, 0)),
            pl.BlockSpec((bs, 2, HW), lambda: (0, 0, 0)),
        ],
        out_specs=pl.BlockSpec((bs, 4, HW), lambda: (0, 0, 0)),
        out_shape=jax.ShapeDtypeStruct((bs, 4, HW), jnp.float32),
    )(thres, rout, dout)

    outs = pl.pallas_call(
        _dense_body,
        grid=(bs,),
        in_specs=[
            pl.BlockSpec((1, C, HW), lambda b: (b, 0, 0)),
            pl.BlockSpec((1, C, HW), lambda b: (b, 0, 0)),
            pl.BlockSpec((1, 4, HW), lambda b: (b, 0, 0)),
        ],
        out_specs=[pl.BlockSpec((1, 1, C), lambda b: (b, 0, 0))] * 4,
        out_shape=[jax.ShapeDtypeStruct((bs, 1, C), jnp.float32)] * 4,
    )(rfea, dfea, w)

    shape = (bs, C, 1, 1)
    rfg_p, rbg_p, dfg_p, dbg_p = outs
    return (rfg_p.reshape(shape), rbg_p.reshape(shape),
            dfg_p.reshape(shape), dbg_p.reshape(shape))


def kernel(res_fea, dinov2_fea, res_out, dinov2_out, fg_thres, bg_thres):
    thres = jnp.stack([jnp.asarray(fg_thres, jnp.float32),
                       jnp.asarray(bg_thres, jnp.float32)])
    return _run(res_fea, dinov2_fea, res_out, dinov2_out, thres)
